# Initial kernel scaffold; baseline (speedup 1.0000x reference)
import jax
import jax.numpy as jnp
from jax.experimental import pallas as pl


def kernel(x, table, W, b, gamma, beta, pos_table, block_table, inner_table):
    raise NotImplementedError("write your pallas kernel here")


# speedup vs baseline: 1.7013x; 1.7013x over previous
import functools

import jax
import jax.numpy as jnp
from jax import lax
from jax.experimental import pallas as pl
from jax.experimental.pallas import tpu as pltpu
from jax.experimental.pallas import tpu_sc as plsc

_VOCAB = 100000
_EMB = 128
_HID = 768
_N_BLOCKS = 16
_BLOCK_SIZE = 128
_B = 32
_T = 2048
_EPS = 1e-12

_TOKENS = _B * _T
_CHUNK = 128
_NW = 32
_PER_W = _TOKENS // _NW
_N_CHUNKS = _PER_W // _CHUNK


def _sc_gather(x2d, table):
    mesh = plsc.VectorSubcoreMesh(core_axis_name="c", subcore_axis_name="s")

    @functools.partial(
        pl.kernel,
        mesh=mesh,
        out_type=jax.ShapeDtypeStruct((_TOKENS, _EMB), jnp.float32),
        scratch_types=[
            pltpu.VMEM((_N_CHUNKS, _CHUNK), jnp.int32),
            pltpu.VMEM((2, _CHUNK, _EMB), jnp.float32),
            pltpu.SemaphoreType.DMA,
            pltpu.SemaphoreType.DMA,
            pltpu.SemaphoreType.DMA,
        ],
    )
    def k(x_hbm, table_hbm, out_hbm, idx_v, rows_v, gsem0, gsem1, wsem):
        wid = lax.axis_index("s") * 2 + lax.axis_index("c")
        row0 = wid * _N_CHUNKS
        pltpu.sync_copy(x_hbm.at[pl.ds(row0, _N_CHUNKS), :], idx_v)

        gsems = [gsem0, gsem1]

        def start_gather(c, buf):
            return pltpu.async_copy(
                table_hbm.at[idx_v.at[c]], rows_v.at[buf], gsems[buf]
            )

        handles = [None, None]
        handles[0] = start_gather(0, 0)
        for c in range(_N_CHUNKS):
            buf = c % 2
            if c + 1 < _N_CHUNKS:
                handles[1 - buf] = start_gather(c + 1, 1 - buf)
            handles[buf].wait()
            out = pltpu.async_copy(
                rows_v.at[buf],
                out_hbm.at[pl.ds((wid * _PER_W) + c * _CHUNK, _CHUNK), :],
                wsem,
            )
            out.wait()

    return k(x2d, table)


def _combine_bias(pos_table, block_table, inner_table, b):
    def body(pos_ref, blk_ref, inner_ref, b_ref, out_ref):
        out_ref[...] = (
            pos_ref[...] + blk_ref[0] + inner_ref[...] + b_ref[...]
        )

    return pl.pallas_call(
        body,
        grid=(_N_BLOCKS,),
        in_specs=[
            pl.BlockSpec((_BLOCK_SIZE, _HID), lambda i: (i, 0)),
            pl.BlockSpec((1, 1, _HID), lambda i: (i, 0, 0)),
            pl.BlockSpec((_BLOCK_SIZE, _HID), lambda i: (0, 0)),
            pl.BlockSpec((1, _HID), lambda i: (0, 0)),
        ],
        out_specs=pl.BlockSpec((_BLOCK_SIZE, _HID), lambda i: (i, 0)),
        out_shape=jax.ShapeDtypeStruct((_T, _HID), jnp.float32),
    )(pos_table, block_table.reshape(_N_BLOCKS, 1, _HID), inner_table,
      b.reshape(1, _HID))


def _proj_ln(emb, W, bias, gamma, beta):
    n_steps = _TOKENS // _BLOCK_SIZE

    def body(emb_ref, w_ref, bias_ref, gamma_ref, beta_ref, out_ref):
        i = pl.program_id(0)
        h = jnp.dot(emb_ref[...], w_ref[...],
                    preferred_element_type=jnp.float32)
        pos = bias_ref[pl.ds((i % _N_BLOCKS) * _BLOCK_SIZE, _BLOCK_SIZE), :]
        h = h + pos
        u = jnp.mean(h, axis=-1, keepdims=True)
        d = h - u
        s = jnp.mean(d * d, axis=-1, keepdims=True)
        out_ref[...] = gamma_ref[...] * (d * lax.rsqrt(s + _EPS)) + beta_ref[...]

    return pl.pallas_call(
        body,
        grid=(n_steps,),
        in_specs=[
            pl.BlockSpec((_BLOCK_SIZE, _EMB), lambda i: (i, 0)),
            pl.BlockSpec((_EMB, _HID), lambda i: (0, 0)),
            pl.BlockSpec((_T, _HID), lambda i: (0, 0)),
            pl.BlockSpec((1, _HID), lambda i: (0, 0)),
            pl.BlockSpec((1, _HID), lambda i: (0, 0)),
        ],
        out_specs=pl.BlockSpec((_BLOCK_SIZE, _HID), lambda i: (i, 0)),
        out_shape=jax.ShapeDtypeStruct((_TOKENS, _HID), jnp.float32),
    )(emb, W, bias, gamma.reshape(1, _HID), beta.reshape(1, _HID))


def kernel(x, table, W, b, gamma, beta, pos_table, block_table, inner_table):
    x2d = x.reshape(_TOKENS // _CHUNK, _CHUNK)
    emb = _sc_gather(x2d, table)
    bias = _combine_bias(pos_table, block_table, inner_table, b)
    out = _proj_ln(emb, W, bias, gamma, beta)
    return out.reshape(_B, _T, _HID)


# speedup vs baseline: 2.6660x; 1.5670x over previous
import functools

import jax
import jax.numpy as jnp
from jax import lax
from jax.experimental import pallas as pl
from jax.experimental.pallas import tpu as pltpu
from jax.experimental.pallas import tpu_sc as plsc

_VOCAB = 100000
_EMB = 128
_HID = 768
_N_BLOCKS = 16
_BLOCK_SIZE = 128
_B = 32
_T = 2048
_EPS = 1e-12

_TOKENS = _B * _T
_CHUNK = 128
_NW = 32
_PER_W = _TOKENS // _NW
_N_CHUNKS = _PER_W // _CHUNK


def _sc_gather(x2d, table):
    mesh = plsc.VectorSubcoreMesh(core_axis_name="c", subcore_axis_name="s")

    @functools.partial(
        pl.kernel,
        mesh=mesh,
        out_type=jax.ShapeDtypeStruct((_TOKENS, _EMB), jnp.float32),
        scratch_types=[
            pltpu.VMEM((_N_CHUNKS, _CHUNK), jnp.int32),
            pltpu.VMEM((2, _CHUNK, _EMB), jnp.float32),
            pltpu.SemaphoreType.DMA,
            pltpu.SemaphoreType.DMA,
            pltpu.SemaphoreType.DMA,
        ],
    )
    def k(x_hbm, table_hbm, out_hbm, idx_v, rows_v, gsem0, gsem1, wsem):
        wid = lax.axis_index("s") * 2 + lax.axis_index("c")
        row0 = wid * _N_CHUNKS
        pltpu.sync_copy(x_hbm.at[pl.ds(row0, _N_CHUNKS), :], idx_v)

        gsems = [gsem0, gsem1]

        def start_gather(c, buf):
            return pltpu.async_copy(
                table_hbm.at[idx_v.at[c]], rows_v.at[buf], gsems[buf]
            )

        handles = [None, None]
        handles[0] = start_gather(0, 0)
        for c in range(_N_CHUNKS):
            buf = c % 2
            if c + 1 < _N_CHUNKS:
                handles[1 - buf] = start_gather(c + 1, 1 - buf)
            handles[buf].wait()
            out = pltpu.async_copy(
                rows_v.at[buf],
                out_hbm.at[pl.ds((wid * _PER_W) + c * _CHUNK, _CHUNK), :],
                wsem,
            )
            out.wait()

    return k(x2d, table)


def _combine_bias(pos_table, block_table, inner_table, b):
    def body(pos_ref, blk_ref, inner_ref, b_ref, out_ref):
        out_ref[...] = (
            pos_ref[...] + blk_ref[0] + inner_ref[...] + b_ref[...]
        )

    return pl.pallas_call(
        body,
        grid=(_N_BLOCKS,),
        in_specs=[
            pl.BlockSpec((_BLOCK_SIZE, _HID), lambda i: (i, 0)),
            pl.BlockSpec((1, 1, _HID), lambda i: (i, 0, 0)),
            pl.BlockSpec((_BLOCK_SIZE, _HID), lambda i: (0, 0)),
            pl.BlockSpec((1, _HID), lambda i: (0, 0)),
        ],
        out_specs=pl.BlockSpec((_BLOCK_SIZE, _HID), lambda i: (i, 0)),
        out_shape=jax.ShapeDtypeStruct((_T, _HID), jnp.float32),
    )(pos_table, block_table.reshape(_N_BLOCKS, 1, _HID), inner_table,
      b.reshape(1, _HID))


_TB = 256


def _proj_ln(emb, W, bias, gamma, beta):
    n_steps = _TOKENS // _TB
    blocks_per_seq = _T // _TB

    def body(emb_ref, w_ref, bias_ref, gamma_ref, beta_ref, out_ref):
        i = pl.program_id(0)
        a = emb_ref[...].astype(jnp.bfloat16)
        w = w_ref[...].astype(jnp.bfloat16)
        h = jnp.dot(a, w, preferred_element_type=jnp.float32)
        pos = bias_ref[pl.ds((i % blocks_per_seq) * _TB, _TB), :]
        h = h + pos
        u = jnp.mean(h, axis=-1, keepdims=True)
        d = h - u
        s = jnp.mean(d * d, axis=-1, keepdims=True)
        out_ref[...] = gamma_ref[...] * (d * lax.rsqrt(s + _EPS)) + beta_ref[...]

    return pl.pallas_call(
        body,
        grid=(n_steps,),
        in_specs=[
            pl.BlockSpec((_TB, _EMB), lambda i: (i, 0)),
            pl.BlockSpec((_EMB, _HID), lambda i: (0, 0)),
            pl.BlockSpec((_T, _HID), lambda i: (0, 0)),
            pl.BlockSpec((1, _HID), lambda i: (0, 0)),
            pl.BlockSpec((1, _HID), lambda i: (0, 0)),
        ],
        out_specs=pl.BlockSpec((_TB, _HID), lambda i: (i, 0)),
        out_shape=jax.ShapeDtypeStruct((_TOKENS, _HID), jnp.float32),
    )(emb, W, bias, gamma.reshape(1, _HID), beta.reshape(1, _HID))


def kernel(x, table, W, b, gamma, beta, pos_table, block_table, inner_table):
    x2d = x.reshape(_TOKENS // _CHUNK, _CHUNK)
    emb = _sc_gather(x2d, table)
    bias = _combine_bias(pos_table, block_table, inner_table, b)
    out = _proj_ln(emb, W, bias, gamma, beta)
    return out.reshape(_B, _T, _HID)
